# baseline (speedup 1.0000x reference)
import functools

import jax
import jax.numpy as jnp
from jax.experimental import pallas as pl
from jax.experimental.pallas import tpu as pltpu

_TC = 512
_K = 16
_PAD = 128


def _fused_body(x_ref, w_ref, idx_ref, wz_ref, sz_ref, cv_ref, ci_ref, s_ref):
    i = pl.program_id(0)
    n_chunks = pl.num_programs(0)
    xb = x_ref[...]
    b, tc, d = xb.shape
    z = jax.lax.dot_general(
        xb.reshape(b * tc, d), w_ref[...],
        (((1,), (0,)), ((), ())),
        preferred_element_type=jnp.float32,
    )
    et = jnp.tanh(z).reshape(b, tc)
    part = jnp.sum(xb, axis=1)

    @pl.when(i == 0)
    def _():
        s_ref[...] = part

    @pl.when(i > 0)
    def _():
        s_ref[...] += part

    iota = jax.lax.broadcasted_iota(jnp.int32, (b, tc), 1)
    k_iota = jax.lax.broadcasted_iota(jnp.int32, (b, _K), 1)
    cur = et
    vals = jnp.full((b, _K), -2.0, jnp.float32)
    idxs = jnp.zeros((b, _K), jnp.int32)
    for k in range(_K):
        v = jnp.max(cur, axis=1, keepdims=True)
        am = jnp.min(jnp.where(cur == v, iota, tc), axis=1, keepdims=True)
        vals = jnp.where(k_iota == k, v, vals)
        idxs = jnp.where(k_iota == k, am, idxs)
        cur = jnp.where(iota == am, -2.0, cur)
    cv_ref[:, pl.ds(i, 1), :] = vals[:, None, :]
    ci_ref[:, pl.ds(i, 1), :] = (idxs + i * tc)[:, None, :]

    @pl.when(i == n_chunks - 1)
    def _():
        nc = n_chunks * _K
        cv = cv_ref[...].reshape(b, nc)
        ci = ci_ref[...].reshape(b, nc)
        m_iota = jax.lax.broadcasted_iota(jnp.int32, (b, nc), 1)
        p_iota = jax.lax.broadcasted_iota(jnp.int32, (b, _PAD), 1)
        cur2 = cv
        gv = jnp.full((b, _PAD), -2.0, jnp.float32)
        gi = jnp.zeros((b, _PAD), jnp.int32)
        for k in range(_K):
            v = jnp.max(cur2, axis=1, keepdims=True)
            am = jnp.min(jnp.where(cur2 == v, m_iota, nc), axis=1, keepdims=True)
            gidx = jnp.sum(
                jnp.where(m_iota == am, ci, 0), axis=1, keepdims=True)
            gv = jnp.where(p_iota == k, v, gv)
            gi = jnp.where(p_iota == k, gidx, gi)
            cur2 = jnp.where(m_iota == am, -2.0, cur2)
        thresh = jnp.sum(jnp.where(p_iota == 4, gv, 0.0), axis=1, keepdims=True)
        w = jnp.where(gv >= thresh, jnp.exp(gv) - 1.0, 0.0)
        zden = n_chunks * tc + jnp.sum(w, axis=1, keepdims=True)
        idx_ref[...] = gi
        wz_ref[...] = w / zden
        sz_ref[...] = s_ref[...] / zden


def _gather_body(idx_ref, wz_ref, *refs):
    b = pl.program_id(0)
    x_refs = refs[:_K]
    sz_ref, out_ref = refs[_K], refs[_K + 1]
    acc = sz_ref[0]
    for k in range(_K):
        m = idx_ref[b, k] % 8
        acc = acc + wz_ref[b, k] * x_refs[k][0, pl.ds(m, 1), :]
    out_ref[0] = acc


def kernel(x, W):
    B, T, D = x.shape
    n_chunks = T // _TC

    idx, wz, sz = pl.pallas_call(
        _fused_body,
        grid=(n_chunks,),
        in_specs=[
            pl.BlockSpec((B, _TC, D), lambda i: (0, i, 0)),
            pl.BlockSpec((D, 1), lambda i: (0, 0)),
        ],
        out_specs=[
            pl.BlockSpec((B, _PAD), lambda i: (0, 0)),
            pl.BlockSpec((B, _PAD), lambda i: (0, 0)),
            pl.BlockSpec((B, D), lambda i: (0, 0)),
        ],
        out_shape=[
            jax.ShapeDtypeStruct((B, _PAD), jnp.int32),
            jax.ShapeDtypeStruct((B, _PAD), jnp.float32),
            jax.ShapeDtypeStruct((B, D), jnp.float32),
        ],
        scratch_shapes=[
            pltpu.VMEM((B, n_chunks, _K), jnp.float32),
            pltpu.VMEM((B, n_chunks, _K), jnp.int32),
            pltpu.VMEM((B, D), jnp.float32),
        ],
    )(x, W)

    out3 = pl.pallas_call(
        _gather_body,
        grid_spec=pltpu.PrefetchScalarGridSpec(
            num_scalar_prefetch=2,
            grid=(B,),
            in_specs=[
                pl.BlockSpec(
                    (1, 8, D),
                    functools.partial(
                        lambda kk, b, idx, wz: (b, idx[b, kk] // 8, 0), k
                    ),
                )
                for k in range(_K)
            ] + [pl.BlockSpec((1, 1, D), lambda b, idx, wz: (b, 0, 0))],
            out_specs=pl.BlockSpec((1, 1, D), lambda b, idx, wz: (b, 0, 0)),
        ),
        out_shape=jax.ShapeDtypeStruct((B, 1, D), jnp.float32),
    )(idx, wz, *([x] * _K), sz.reshape(B, 1, D))

    return out3.reshape(B, D)


# speedup vs baseline: 1.7204x; 1.7204x over previous
import functools

import jax
import jax.numpy as jnp
from jax.experimental import pallas as pl
from jax.experimental.pallas import tpu as pltpu

_TC = 256
_K = 16
_PAD = 128


def _pass1_body(x_ref, w_ref, et_ref, s_ref):
    i = pl.program_id(0)
    xb = x_ref[...]
    b, tc, d = xb.shape
    z = jax.lax.dot_general(
        xb.reshape(b * tc, d), w_ref[...],
        (((1,), (0,)), ((), ())),
        preferred_element_type=jnp.float32,
    )
    et_ref[...] = jnp.tanh(z).reshape(b, tc)
    part = jnp.sum(xb, axis=1)

    @pl.when(i == 0)
    def _():
        s_ref[...] = part

    @pl.when(i > 0)
    def _():
        s_ref[...] += part


def _topk_body(et_ref, s_ref, idx_ref, wz_ref, sz_ref):
    et = et_ref[...]
    b, t = et.shape
    iota = jax.lax.broadcasted_iota(jnp.int32, (b, t), 1)
    k_iota = jax.lax.broadcasted_iota(jnp.int32, (b, _PAD), 1)
    cur = et
    vals = jnp.full((b, _PAD), -2.0, jnp.float32)
    idxs = jnp.zeros((b, _PAD), jnp.int32)
    for k in range(_K):
        v = jnp.max(cur, axis=1, keepdims=True)
        am = jnp.min(jnp.where(cur == v, iota, t), axis=1, keepdims=True)
        vals = jnp.where(k_iota == k, v, vals)
        idxs = jnp.where(k_iota == k, am, idxs)
        cur = jnp.where(iota == am, -2.0, cur)
    thresh = jnp.sum(jnp.where(k_iota == 4, vals, 0.0), axis=1, keepdims=True)
    w = jnp.where(vals >= thresh, jnp.exp(vals) - 1.0, 0.0)
    zden = t + jnp.sum(w, axis=1, keepdims=True)
    idx_ref[...] = idxs
    wz_ref[...] = w / zden
    sz_ref[...] = s_ref[...] / zden


def _gather_body(idx_ref, wz_ref, *refs):
    b = pl.program_id(0)
    x_refs = refs[:_K]
    sz_ref, out_ref = refs[_K], refs[_K + 1]
    acc = sz_ref[0]
    for k in range(_K):
        m = idx_ref[b, k] % 8
        acc = acc + wz_ref[b, k] * x_refs[k][0, pl.ds(m, 1), :]
    out_ref[0] = acc


def kernel(x, W):
    B, T, D = x.shape
    n_chunks = T // _TC

    et, S = pl.pallas_call(
        _pass1_body,
        grid=(n_chunks,),
        in_specs=[
            pl.BlockSpec((B, _TC, D), lambda i: (0, i, 0)),
            pl.BlockSpec((D, 1), lambda i: (0, 0)),
        ],
        out_specs=[
            pl.BlockSpec((B, _TC), lambda i: (0, i)),
            pl.BlockSpec((B, D), lambda i: (0, 0)),
        ],
        out_shape=[
            jax.ShapeDtypeStruct((B, T), jnp.float32),
            jax.ShapeDtypeStruct((B, D), jnp.float32),
        ],
    )(x, W)

    idx, wz, sz = pl.pallas_call(
        _topk_body,
        in_specs=[
            pl.BlockSpec((B, T), lambda: (0, 0)),
            pl.BlockSpec((B, D), lambda: (0, 0)),
        ],
        out_specs=[
            pl.BlockSpec((B, _PAD), lambda: (0, 0)),
            pl.BlockSpec((B, _PAD), lambda: (0, 0)),
            pl.BlockSpec((B, D), lambda: (0, 0)),
        ],
        out_shape=[
            jax.ShapeDtypeStruct((B, _PAD), jnp.int32),
            jax.ShapeDtypeStruct((B, _PAD), jnp.float32),
            jax.ShapeDtypeStruct((B, D), jnp.float32),
        ],
    )(et, S)

    out3 = pl.pallas_call(
        _gather_body,
        grid_spec=pltpu.PrefetchScalarGridSpec(
            num_scalar_prefetch=2,
            grid=(B,),
            in_specs=[
                pl.BlockSpec(
                    (1, 8, D),
                    functools.partial(
                        lambda kk, b, idx, wz: (b, idx[b, kk] // 8, 0), k
                    ),
                )
                for k in range(_K)
            ] + [pl.BlockSpec((1, 1, D), lambda b, idx, wz: (b, 0, 0))],
            out_specs=pl.BlockSpec((1, 1, D), lambda b, idx, wz: (b, 0, 0)),
        ),
        out_shape=jax.ShapeDtypeStruct((B, 1, D), jnp.float32),
    )(idx, wz, *([x] * _K), sz.reshape(B, 1, D))

    return out3.reshape(B, D)


# speedup vs baseline: 2.3435x; 1.3622x over previous
import functools

import jax
import jax.numpy as jnp
from jax.experimental import pallas as pl
from jax.experimental.pallas import tpu as pltpu

_TC = 512
_K = 16
_PAD = 128


def _pass1_body(x_ref, w_ref, et_ref, s_ref):
    i = pl.program_id(0)
    xb = x_ref[...]
    b, tc, d = xb.shape
    z = jax.lax.dot_general(
        xb.reshape(b * tc, d), w_ref[...],
        (((1,), (0,)), ((), ())),
        preferred_element_type=jnp.float32,
    )
    et_ref[...] = jnp.tanh(z).reshape(b, tc)
    part = jnp.sum(xb, axis=1)

    @pl.when(i == 0)
    def _():
        s_ref[...] = part

    @pl.when(i > 0)
    def _():
        s_ref[...] += part


def _topk_body(et_ref, s_ref, idx_ref, wz_ref, sz_ref):
    et = et_ref[...]
    b, t = et.shape
    iota = jax.lax.broadcasted_iota(jnp.int32, (b, t), 1)
    k_iota = jax.lax.broadcasted_iota(jnp.int32, (b, _PAD), 1)
    cur = et
    vals = jnp.full((b, _PAD), -2.0, jnp.float32)
    idxs = jnp.zeros((b, _PAD), jnp.int32)
    for k in range(_K):
        v = jnp.max(cur, axis=1, keepdims=True)
        am = jnp.min(jnp.where(cur == v, iota, t), axis=1, keepdims=True)
        vals = jnp.where(k_iota == k, v, vals)
        idxs = jnp.where(k_iota == k, am, idxs)
        cur = jnp.where(iota == am, -2.0, cur)
    thresh = jnp.sum(jnp.where(k_iota == 4, vals, 0.0), axis=1, keepdims=True)
    w = jnp.where(vals >= thresh, jnp.exp(vals) - 1.0, 0.0)
    zden = t + jnp.sum(w, axis=1, keepdims=True)
    idx_ref[...] = idxs
    wz_ref[...] = w / zden
    sz_ref[...] = s_ref[...] / zden


def _tail_body(et_ref, s_ref, x_hbm, out_ref, iscr, wscr, rows, sem):
    et = et_ref[...]
    b, t = et.shape
    iota = jax.lax.broadcasted_iota(jnp.int32, (b, t), 1)
    k_iota = jax.lax.broadcasted_iota(jnp.int32, (b, _PAD), 1)
    cur = et
    vals = jnp.full((b, _PAD), -2.0, jnp.float32)
    idxs = jnp.zeros((b, _PAD), jnp.int32)
    for k in range(_K):
        v = jnp.max(cur, axis=1, keepdims=True)
        am = jnp.min(jnp.where(cur == v, iota, t), axis=1, keepdims=True)
        vals = jnp.where(k_iota == k, v, vals)
        idxs = jnp.where(k_iota == k, am, idxs)
        cur = jnp.where(iota == am, -2.0, cur)
    thresh = jnp.sum(jnp.where(k_iota == 4, vals, 0.0), axis=1, keepdims=True)
    w = jnp.where(vals >= thresh, jnp.exp(vals) - 1.0, 0.0)
    zden = t + jnp.sum(w, axis=1, keepdims=True)
    iscr[...] = idxs
    wscr[...] = jnp.where(k_iota == _K, zden, w)
    for bb in range(b):
        for k in range(_K):
            pltpu.make_async_copy(
                x_hbm.at[bb, pl.ds(iscr[bb, k], 1), :],
                rows.at[pl.ds(bb * _K + k, 1), :], sem).start()
    for bb in range(b):
        for k in range(_K):
            pltpu.make_async_copy(
                x_hbm.at[bb, pl.ds(iscr[bb, k], 1), :],
                rows.at[pl.ds(bb * _K + k, 1), :], sem).wait()
    for bb in range(b):
        acc = s_ref[pl.ds(bb, 1), :]
        for k in range(_K):
            acc = acc + wscr[bb, k] * rows[pl.ds(bb * _K + k, 1), :]
        out_ref[pl.ds(bb, 1), :] = acc / wscr[bb, _K]


def kernel(x, W):
    B, T, D = x.shape
    n_chunks = T // _TC

    et, S = pl.pallas_call(
        _pass1_body,
        grid=(n_chunks,),
        in_specs=[
            pl.BlockSpec((B, _TC, D), lambda i: (0, i, 0)),
            pl.BlockSpec((D, 1), lambda i: (0, 0)),
        ],
        out_specs=[
            pl.BlockSpec((B, _TC), lambda i: (0, i)),
            pl.BlockSpec((B, D), lambda i: (0, 0)),
        ],
        out_shape=[
            jax.ShapeDtypeStruct((B, T), jnp.float32),
            jax.ShapeDtypeStruct((B, D), jnp.float32),
        ],
    )(x, W)

    out = pl.pallas_call(
        _tail_body,
        in_specs=[
            pl.BlockSpec((B, T), lambda: (0, 0)),
            pl.BlockSpec((B, D), lambda: (0, 0)),
            pl.BlockSpec(memory_space=pl.ANY),
        ],
        out_specs=pl.BlockSpec((B, D), lambda: (0, 0)),
        out_shape=jax.ShapeDtypeStruct((B, D), jnp.float32),
        scratch_shapes=[
            pltpu.VMEM((B, _PAD), jnp.int32),
            pltpu.VMEM((B, _PAD), jnp.float32),
            pltpu.VMEM((B * _K, D), jnp.float32),
            pltpu.SemaphoreType.DMA,
        ],
    )(et, S, x)

    return out
